# baseline (speedup 1.0000x reference)
import functools

import jax
import jax.numpy as jnp
from jax import lax
from jax.experimental import pallas as pl
from jax.experimental.pallas import tpu as pltpu
from jax.experimental.pallas import tpu_sc as plsc

_BATCH = 16384
_C = 32
_NW = 32
_BPW = _BATCH // _NW
_CH = 256
_LN2 = 0.6931471805599453


def _ln(x):
    bits = lax.bitcast_convert_type(x, jnp.int32)
    e = jnp.right_shift(bits, 23) - 127
    m_bits = jnp.bitwise_or(jnp.bitwise_and(bits, 0x007FFFFF), 0x3F800000)
    m = lax.bitcast_convert_type(m_bits, jnp.float32)
    s = (m - 1.0) / (m + 1.0)
    s2 = s * s
    p = 2.0 * s * (1.0 + s2 * (1.0 / 3.0 + s2 * (0.2 + s2 * (1.0 / 7.0 + s2 * (1.0 / 9.0)))))
    return e.astype(jnp.float32) * _LN2 + p


def _sigmoid04(x):
    return 0.4 / (1.0 + jnp.exp(-x))


def _body(uid_hbm, qid_hbm, theta_hbm, slip_hbm, guess_hbm, qtab_hbm, out_hbm,
          uid_v, qid_v, urow_v, qrow_v, theta_v, qtab_v, slip_v, guess_v, out_v,
          sem_t, sem_q, sem_s, sem_g):
    wid = lax.axis_index("s") * 2 + lax.axis_index("c")
    base = wid * _BPW

    pltpu.sync_copy(uid_hbm.at[pl.ds(base, _BPW)], uid_v)
    pltpu.sync_copy(qid_hbm.at[pl.ds(base, _BPW)], qid_v)

    cp_s = pltpu.async_copy(slip_hbm.at[qid_v], slip_v, sem_s)
    cp_g = pltpu.async_copy(guess_hbm.at[qid_v], guess_v, sem_g)

    lanes = lax.iota(jnp.int32, 16)

    def rows(i, carry):
        u = uid_v[pl.ds(i * 16, 16)]
        q = qid_v[pl.ds(i * 16, 16)]
        urow_v[pl.ds(i * 16, 16)] = jnp.right_shift(u, 2)
        qrow_v[pl.ds(i * 16, 16)] = jnp.right_shift(q, 2)
        return carry

    lax.fori_loop(0, _BPW // 16, rows, 0, unroll=4)

    def process(half, carry):
        hbase = half * _CH
        cp_t = pltpu.async_copy(
            theta_hbm.at[urow_v.at[pl.ds(hbase, _CH)]], theta_v, sem_t)
        cp_q = pltpu.async_copy(
            qtab_hbm.at[qrow_v.at[pl.ds(hbase, _CH)]], qtab_v, sem_q)
        cp_t.wait()
        cp_q.wait()

        def chunk(i, carry2):
            e0 = hbase + i * 16
            rws = i * 16 + lanes
            ucol = jnp.bitwise_and(uid_v[pl.ds(e0, 16)], 3) * _C
            qcol = jnp.bitwise_and(qid_v[pl.ds(e0, 16)], 3) * _C
            n = jnp.full((16,), 1.0, jnp.float32)
            for c in range(_C):
                t = plsc.load_gather(theta_v, [rws, ucol + c])
                q = plsc.load_gather(qtab_v, [rws, qcol + c])
                bad = jnp.logical_and(q > 0.5, t <= 0.0)
                n = n * jnp.where(bad, 0.5, 1.0)
            sraw = slip_v[pl.ds(e0, 16)]
            graw = guess_v[pl.ds(e0, 16)]
            a = 1.0 - _sigmoid04(sraw)
            g = jnp.maximum(_sigmoid04(graw), 1e-30)
            out_v[pl.ds(e0, 16)] = jnp.exp(n * _ln(a) + (1.0 - n) * _ln(g))
            return carry2

        lax.fori_loop(0, _CH // 16, chunk, 0)
        return carry

    lax.fori_loop(0, _BPW // _CH, process, 0)
    cp_s.wait()
    cp_g.wait()
    pltpu.sync_copy(out_v, out_hbm.at[pl.ds(base, _BPW)])


@jax.jit
def _dina_sc(uid, qid, theta_p, slip_1d, guess_1d, qtab_p):
    run = pl.kernel(
        _body,
        out_type=jax.ShapeDtypeStruct((_BATCH,), jnp.float32),
        mesh=plsc.VectorSubcoreMesh(core_axis_name="c", subcore_axis_name="s"),
        compiler_params=pltpu.CompilerParams(needs_layout_passes=False),
        scratch_types=[
            pltpu.VMEM((_BPW,), jnp.int32),
            pltpu.VMEM((_BPW,), jnp.int32),
            pltpu.VMEM((_BPW,), jnp.int32),
            pltpu.VMEM((_BPW,), jnp.int32),
            pltpu.VMEM((_CH, 4 * _C), jnp.float32),
            pltpu.VMEM((_CH, 4 * _C), jnp.float32),
            pltpu.VMEM((_BPW,), jnp.float32),
            pltpu.VMEM((_BPW,), jnp.float32),
            pltpu.VMEM((_BPW,), jnp.float32),
            pltpu.SemaphoreType.DMA,
            pltpu.SemaphoreType.DMA,
            pltpu.SemaphoreType.DMA,
            pltpu.SemaphoreType.DMA,
        ],
    )
    return run(uid, qid, theta_p, slip_1d, guess_1d, qtab_p)


def kernel(user_id, question_id, theta_w, slip_w, guess_w, q_table):
    return _dina_sc(
        user_id.astype(jnp.int32),
        question_id.astype(jnp.int32),
        theta_w.reshape(250000, 128),
        slip_w.reshape(-1),
        guess_w.reshape(-1),
        q_table.reshape(25000, 128),
    )


# speedup vs baseline: 3.8627x; 3.8627x over previous
import functools

import jax
import jax.numpy as jnp
from jax import lax
from jax.experimental import pallas as pl
from jax.experimental.pallas import tpu as pltpu
from jax.experimental.pallas import tpu_sc as plsc

_BATCH = 16384
_C = 32
_NW = 32
_BPW = _BATCH // _NW
_LN2 = 0.6931471805599453


def _pack_body(x_ref, o_ref):
    y = lax.bitcast_convert_type(x_ref[...], jnp.int32)
    bit = jnp.bitwise_and(
        lax.shift_right_logical(jnp.negative(y), 31),
        jnp.right_shift(y, 31) + 1,
    )
    cvec = lax.broadcasted_iota(jnp.int32, (_C, 1), 0)
    o_ref[...] = jnp.sum(jnp.left_shift(bit, cvec), axis=0)


def _pack_tc(xt, n, blk):
    nblk = pl.cdiv(n, blk)
    return pl.pallas_call(
        _pack_body,
        grid=(nblk,),
        in_specs=[pl.BlockSpec((_C, blk), lambda i: (0, i))],
        out_specs=pl.BlockSpec((blk,), lambda i: (i,)),
        out_shape=jax.ShapeDtypeStruct((n,), jnp.int32),
    )(xt)


def _ln(x):
    bits = lax.bitcast_convert_type(x, jnp.int32)
    e = jnp.right_shift(bits, 23) - 127
    m_bits = jnp.bitwise_or(jnp.bitwise_and(bits, 0x007FFFFF), 0x3F800000)
    m = lax.bitcast_convert_type(m_bits, jnp.float32)
    s = (m - 1.0) / (m + 1.0)
    s2 = s * s
    p = 2.0 * s * (1.0 + s2 * (1.0 / 3.0 + s2 * (0.2 + s2 * (1.0 / 7.0 + s2 * (1.0 / 9.0)))))
    return e.astype(jnp.float32) * _LN2 + p


def _sigmoid04(x):
    return 0.4 / (1.0 + jnp.exp(-x))


def _popcount(v):
    v = v - jnp.bitwise_and(lax.shift_right_logical(v, 1), 0x55555555)
    v = jnp.bitwise_and(v, 0x33333333) + jnp.bitwise_and(
        lax.shift_right_logical(v, 2), 0x33333333)
    v = jnp.bitwise_and(v + lax.shift_right_logical(v, 4), 0x0F0F0F0F)
    return lax.shift_right_logical(v * 0x01010101, 24)


def _body(uid_hbm, qid_hbm, tbits_hbm, qbits_hbm, slip_hbm, guess_hbm, out_hbm,
          uid_v, qid_v, tb_v, qb_v, slip_v, guess_v, out_v,
          sem_t, sem_q, sem_s, sem_g):
    wid = lax.axis_index("s") * 2 + lax.axis_index("c")
    base = wid * _BPW

    pltpu.sync_copy(uid_hbm.at[pl.ds(base, _BPW)], uid_v)
    pltpu.sync_copy(qid_hbm.at[pl.ds(base, _BPW)], qid_v)

    cp_t = pltpu.async_copy(tbits_hbm.at[uid_v], tb_v, sem_t)
    cp_q = pltpu.async_copy(qbits_hbm.at[qid_v], qb_v, sem_q)
    cp_s = pltpu.async_copy(slip_hbm.at[qid_v], slip_v, sem_s)
    cp_g = pltpu.async_copy(guess_hbm.at[qid_v], guess_v, sem_g)
    cp_t.wait()
    cp_q.wait()
    cp_s.wait()
    cp_g.wait()

    def chunk(i, carry):
        tb = tb_v[pl.ds(i * 16, 16)]
        qb = qb_v[pl.ds(i * 16, 16)]
        bad = jnp.bitwise_and(qb, jnp.bitwise_not(tb))
        k = _popcount(bad)
        n = lax.bitcast_convert_type(jnp.left_shift(127 - k, 23), jnp.float32)
        sraw = slip_v[pl.ds(i * 16, 16)]
        graw = guess_v[pl.ds(i * 16, 16)]
        a = 1.0 - _sigmoid04(sraw)
        g = jnp.maximum(_sigmoid04(graw), 1e-30)
        out_v[pl.ds(i * 16, 16)] = jnp.exp(n * _ln(a) + (1.0 - n) * _ln(g))
        return carry

    lax.fori_loop(0, _BPW // 16, chunk, 0)
    pltpu.sync_copy(out_v, out_hbm.at[pl.ds(base, _BPW)])


@jax.jit
def _dina(uid, qid, theta_t, slip_1d, guess_1d, qtab_t):
    tbits = _pack_tc(theta_t, theta_t.shape[1], 8192)
    qbits = _pack_tc(qtab_t, qtab_t.shape[1], 4096)
    run = pl.kernel(
        _body,
        out_type=jax.ShapeDtypeStruct((_BATCH,), jnp.float32),
        mesh=plsc.VectorSubcoreMesh(core_axis_name="c", subcore_axis_name="s"),
        compiler_params=pltpu.CompilerParams(needs_layout_passes=False),
        scratch_types=[
            pltpu.VMEM((_BPW,), jnp.int32),
            pltpu.VMEM((_BPW,), jnp.int32),
            pltpu.VMEM((_BPW,), jnp.int32),
            pltpu.VMEM((_BPW,), jnp.int32),
            pltpu.VMEM((_BPW,), jnp.float32),
            pltpu.VMEM((_BPW,), jnp.float32),
            pltpu.VMEM((_BPW,), jnp.float32),
            pltpu.SemaphoreType.DMA,
            pltpu.SemaphoreType.DMA,
            pltpu.SemaphoreType.DMA,
            pltpu.SemaphoreType.DMA,
        ],
    )
    return run(uid, qid, tbits, qbits, slip_1d, guess_1d)


def kernel(user_id, question_id, theta_w, slip_w, guess_w, q_table):
    return _dina(
        user_id.astype(jnp.int32),
        question_id.astype(jnp.int32),
        theta_w.T,
        slip_w.reshape(-1),
        guess_w.reshape(-1),
        q_table.T,
    )


# speedup vs baseline: 4.0345x; 1.0445x over previous
import functools

import jax
import jax.numpy as jnp
from jax import lax
from jax.experimental import pallas as pl
from jax.experimental.pallas import tpu as pltpu
from jax.experimental.pallas import tpu_sc as plsc

_BATCH = 16384
_C = 32
_NW = 32
_BPW = _BATCH // _NW
_LN2 = 0.6931471805599453


def _pack_body(x_ref, o_ref):
    x = x_ref[...]
    bits = jnp.where(x > 0, 1.0, 0.0).astype(jnp.float32)
    c = lax.broadcasted_iota(jnp.int32, (8, _C), 1)
    r = lax.broadcasted_iota(jnp.int32, (8, _C), 0)
    pow2 = jnp.left_shift(1, jnp.bitwise_and(c, 15)).astype(jnp.float32)
    w = jnp.where(jnp.right_shift(c, 4) == r, pow2, 0.0)
    halves = jax.lax.dot_general(
        w, bits, (((1,), (0,)), ((), ())),
        preferred_element_type=jnp.float32)
    lo = halves[0].astype(jnp.int32)
    hi = halves[1].astype(jnp.int32)
    o_ref[...] = jnp.bitwise_or(lo, jnp.left_shift(hi, 16))


def _pack_tc(xt, n, blk):
    nblk = pl.cdiv(n, blk)
    return pl.pallas_call(
        _pack_body,
        grid=(nblk,),
        in_specs=[pl.BlockSpec((_C, blk), lambda i: (0, i))],
        out_specs=pl.BlockSpec((blk,), lambda i: (i,)),
        out_shape=jax.ShapeDtypeStruct((n,), jnp.int32),
    )(xt)


def _ln(x):
    bits = lax.bitcast_convert_type(x, jnp.int32)
    e = jnp.right_shift(bits, 23) - 127
    m_bits = jnp.bitwise_or(jnp.bitwise_and(bits, 0x007FFFFF), 0x3F800000)
    m = lax.bitcast_convert_type(m_bits, jnp.float32)
    s = (m - 1.0) / (m + 1.0)
    s2 = s * s
    p = 2.0 * s * (1.0 + s2 * (1.0 / 3.0 + s2 * (0.2 + s2 * (1.0 / 7.0 + s2 * (1.0 / 9.0)))))
    return e.astype(jnp.float32) * _LN2 + p


def _sigmoid04(x):
    return 0.4 / (1.0 + jnp.exp(-x))


def _popcount(v):
    v = v - jnp.bitwise_and(lax.shift_right_logical(v, 1), 0x55555555)
    v = jnp.bitwise_and(v, 0x33333333) + jnp.bitwise_and(
        lax.shift_right_logical(v, 2), 0x33333333)
    v = jnp.bitwise_and(v + lax.shift_right_logical(v, 4), 0x0F0F0F0F)
    return lax.shift_right_logical(v * 0x01010101, 24)


def _body(uid_hbm, qid_hbm, tbits_hbm, qbits_hbm, slip_hbm, guess_hbm, out_hbm,
          uid_v, qid_v, tb_v, qb_v, slip_v, guess_v, out_v,
          sem_t, sem_q, sem_s, sem_g):
    wid = lax.axis_index("s") * 2 + lax.axis_index("c")
    base = wid * _BPW

    pltpu.sync_copy(uid_hbm.at[pl.ds(base, _BPW)], uid_v)
    pltpu.sync_copy(qid_hbm.at[pl.ds(base, _BPW)], qid_v)

    cp_t = pltpu.async_copy(tbits_hbm.at[uid_v], tb_v, sem_t)
    cp_q = pltpu.async_copy(qbits_hbm.at[qid_v], qb_v, sem_q)
    cp_s = pltpu.async_copy(slip_hbm.at[qid_v], slip_v, sem_s)
    cp_g = pltpu.async_copy(guess_hbm.at[qid_v], guess_v, sem_g)
    cp_t.wait()
    cp_q.wait()
    cp_s.wait()
    cp_g.wait()

    def chunk(i, carry):
        tb = tb_v[pl.ds(i * 16, 16)]
        qb = qb_v[pl.ds(i * 16, 16)]
        bad = jnp.bitwise_and(qb, jnp.bitwise_not(tb))
        k = _popcount(bad)
        n = lax.bitcast_convert_type(jnp.left_shift(127 - k, 23), jnp.float32)
        sraw = slip_v[pl.ds(i * 16, 16)]
        graw = guess_v[pl.ds(i * 16, 16)]
        a = 1.0 - _sigmoid04(sraw)
        g = jnp.maximum(_sigmoid04(graw), 1e-30)
        out_v[pl.ds(i * 16, 16)] = jnp.exp(n * _ln(a) + (1.0 - n) * _ln(g))
        return carry

    lax.fori_loop(0, _BPW // 16, chunk, 0)
    pltpu.sync_copy(out_v, out_hbm.at[pl.ds(base, _BPW)])


@jax.jit
def _dina(uid, qid, theta_t, slip_1d, guess_1d, qtab_t):
    tbits = _pack_tc(theta_t, theta_t.shape[1], 8192)
    qbits = _pack_tc(qtab_t, qtab_t.shape[1], 4096)
    run = pl.kernel(
        _body,
        out_type=jax.ShapeDtypeStruct((_BATCH,), jnp.float32),
        mesh=plsc.VectorSubcoreMesh(core_axis_name="c", subcore_axis_name="s"),
        compiler_params=pltpu.CompilerParams(needs_layout_passes=False),
        scratch_types=[
            pltpu.VMEM((_BPW,), jnp.int32),
            pltpu.VMEM((_BPW,), jnp.int32),
            pltpu.VMEM((_BPW,), jnp.int32),
            pltpu.VMEM((_BPW,), jnp.int32),
            pltpu.VMEM((_BPW,), jnp.float32),
            pltpu.VMEM((_BPW,), jnp.float32),
            pltpu.VMEM((_BPW,), jnp.float32),
            pltpu.SemaphoreType.DMA,
            pltpu.SemaphoreType.DMA,
            pltpu.SemaphoreType.DMA,
            pltpu.SemaphoreType.DMA,
        ],
    )
    return run(uid, qid, tbits, qbits, slip_1d, guess_1d)


def kernel(user_id, question_id, theta_w, slip_w, guess_w, q_table):
    return _dina(
        user_id.astype(jnp.int32),
        question_id.astype(jnp.int32),
        theta_w.T,
        slip_w.reshape(-1),
        guess_w.reshape(-1),
        q_table.T,
    )


# speedup vs baseline: 6.9540x; 1.7236x over previous
import functools

import jax
import jax.numpy as jnp
from jax import lax
from jax.experimental import pallas as pl
from jax.experimental.pallas import tpu as pltpu
from jax.experimental.pallas import tpu_sc as plsc

_BATCH = 16384
_C = 32
_NW = 32
_BPW = _BATCH // _NW
_LN2 = 0.6931471805599453


def _pack_body(x_ref, o_ref):
    x = x_ref[...]
    bits = jnp.where(x > 0, 1.0, 0.0).astype(jnp.float32)
    c = lax.broadcasted_iota(jnp.int32, (8, _C), 1)
    r = lax.broadcasted_iota(jnp.int32, (8, _C), 0)
    pow2 = jnp.left_shift(1, jnp.bitwise_and(c, 15)).astype(jnp.float32)
    w = jnp.where(jnp.right_shift(c, 4) == r, pow2, 0.0)
    halves = jax.lax.dot_general(
        w, bits, (((1,), (0,)), ((), ())),
        preferred_element_type=jnp.float32)
    lo = halves[0].astype(jnp.int32)
    hi = halves[1].astype(jnp.int32)
    o_ref[...] = jnp.bitwise_or(lo, jnp.left_shift(hi, 16))


def _pack_tc(xt, n, blk):
    nblk = pl.cdiv(n, blk)
    return pl.pallas_call(
        _pack_body,
        grid=(nblk,),
        in_specs=[pl.BlockSpec((_C, blk), lambda i: (0, i))],
        out_specs=pl.BlockSpec((blk,), lambda i: (i,)),
        out_shape=jax.ShapeDtypeStruct((n,), jnp.int32),
    )(xt)


def _ln(x):
    bits = lax.bitcast_convert_type(x, jnp.int32)
    e = jnp.right_shift(bits, 23) - 127
    m_bits = jnp.bitwise_or(jnp.bitwise_and(bits, 0x007FFFFF), 0x3F800000)
    m = lax.bitcast_convert_type(m_bits, jnp.float32)
    s = (m - 1.0) / (m + 1.0)
    s2 = s * s
    p = 2.0 * s * (1.0 + s2 * (1.0 / 3.0 + s2 * (0.2 + s2 * (1.0 / 7.0 + s2 * (1.0 / 9.0)))))
    return e.astype(jnp.float32) * _LN2 + p


def _sigmoid04(x):
    return 0.4 / (1.0 + jnp.exp(-x))


def _popcount(v):
    v = v - jnp.bitwise_and(lax.shift_right_logical(v, 1), 0x55555555)
    v = jnp.bitwise_and(v, 0x33333333) + jnp.bitwise_and(
        lax.shift_right_logical(v, 2), 0x33333333)
    v = jnp.bitwise_and(v + lax.shift_right_logical(v, 4), 0x0F0F0F0F)
    return lax.shift_right_logical(v * 0x01010101, 24)


def _body(uid_hbm, qid_hbm, tbits_hbm, qbits_hbm, slip_hbm, guess_hbm, out_hbm,
          uid_v, qid_v, tb_v, qb_v, slip_v, guess_v, out_v,
          sem_t, sem_q, sem_s, sem_g):
    wid = lax.axis_index("s") * 2 + lax.axis_index("c")
    base = wid * _BPW

    pltpu.sync_copy(uid_hbm.at[pl.ds(base, _BPW)], uid_v)
    pltpu.sync_copy(qid_hbm.at[pl.ds(base, _BPW)], qid_v)

    cp_t = pltpu.async_copy(tbits_hbm.at[uid_v], tb_v, sem_t)
    cp_q = pltpu.async_copy(qbits_hbm.at[qid_v], qb_v, sem_q)
    cp_s = pltpu.async_copy(slip_hbm.at[qid_v], slip_v, sem_s)
    cp_g = pltpu.async_copy(guess_hbm.at[qid_v], guess_v, sem_g)
    cp_t.wait()
    cp_q.wait()
    cp_s.wait()
    cp_g.wait()

    def chunk(i, carry):
        tb = tb_v[pl.ds(i * 16, 16)]
        qb = qb_v[pl.ds(i * 16, 16)]
        bad = jnp.bitwise_and(qb, jnp.bitwise_not(tb))
        k = _popcount(bad)
        n = lax.bitcast_convert_type(jnp.left_shift(127 - k, 23), jnp.float32)
        sraw = slip_v[pl.ds(i * 16, 16)]
        graw = guess_v[pl.ds(i * 16, 16)]
        a = 1.0 - _sigmoid04(sraw)
        g = jnp.maximum(_sigmoid04(graw), 1e-30)
        out_v[pl.ds(i * 16, 16)] = jnp.exp(n * _ln(a) + (1.0 - n) * _ln(g))
        return carry

    lax.fori_loop(0, _BPW // 16, chunk, 0)
    pltpu.sync_copy(out_v, out_hbm.at[pl.ds(base, _BPW)])


@jax.jit
def _dina(uid, qid, theta_t, slip_1d, guess_1d, qtab_t):
    tbits = _pack_tc(theta_t, theta_t.shape[1], 32768)
    qbits = _pack_tc(qtab_t, qtab_t.shape[1], 16384)
    run = pl.kernel(
        _body,
        out_type=jax.ShapeDtypeStruct((_BATCH,), jnp.float32),
        mesh=plsc.VectorSubcoreMesh(core_axis_name="c", subcore_axis_name="s"),
        compiler_params=pltpu.CompilerParams(needs_layout_passes=False),
        scratch_types=[
            pltpu.VMEM((_BPW,), jnp.int32),
            pltpu.VMEM((_BPW,), jnp.int32),
            pltpu.VMEM((_BPW,), jnp.int32),
            pltpu.VMEM((_BPW,), jnp.int32),
            pltpu.VMEM((_BPW,), jnp.float32),
            pltpu.VMEM((_BPW,), jnp.float32),
            pltpu.VMEM((_BPW,), jnp.float32),
            pltpu.SemaphoreType.DMA,
            pltpu.SemaphoreType.DMA,
            pltpu.SemaphoreType.DMA,
            pltpu.SemaphoreType.DMA,
        ],
    )
    return run(uid, qid, tbits, qbits, slip_1d, guess_1d)


def kernel(user_id, question_id, theta_w, slip_w, guess_w, q_table):
    return _dina(
        user_id.astype(jnp.int32),
        question_id.astype(jnp.int32),
        theta_w.T,
        slip_w.reshape(-1),
        guess_w.reshape(-1),
        q_table.T,
    )


# speedup vs baseline: 7.5251x; 1.0821x over previous
import functools

import jax
import jax.numpy as jnp
from jax import lax
from jax.experimental import pallas as pl
from jax.experimental.pallas import tpu as pltpu
from jax.experimental.pallas import tpu_sc as plsc

_BATCH = 16384
_C = 32
_NW = 32
_BPW = _BATCH // _NW
_LN2 = 0.6931471805599453


def _pack_body(x_ref, o_ref):
    x = x_ref[...]
    bits = jnp.where(x > 0, 1.0, 0.0).astype(jnp.float32)
    c = lax.broadcasted_iota(jnp.int32, (8, _C), 1)
    r = lax.broadcasted_iota(jnp.int32, (8, _C), 0)
    pow2 = jnp.left_shift(1, jnp.bitwise_and(c, 15)).astype(jnp.float32)
    w = jnp.where(jnp.right_shift(c, 4) == r, pow2, 0.0)
    halves = jax.lax.dot_general(
        w, bits, (((1,), (0,)), ((), ())),
        preferred_element_type=jnp.float32)
    lo = halves[0].astype(jnp.int32)
    hi = halves[1].astype(jnp.int32)
    o_ref[...] = jnp.bitwise_or(lo, jnp.left_shift(hi, 16))


def _pack_tc(xt, n, blk):
    nblk = pl.cdiv(n, blk)
    return pl.pallas_call(
        _pack_body,
        grid=(nblk,),
        in_specs=[pl.BlockSpec((_C, blk), lambda i: (0, i))],
        out_specs=pl.BlockSpec((blk,), lambda i: (i,)),
        out_shape=jax.ShapeDtypeStruct((n,), jnp.int32),
    )(xt)


def _ln(x):
    bits = lax.bitcast_convert_type(x, jnp.int32)
    e = jnp.right_shift(bits, 23) - 127
    m_bits = jnp.bitwise_or(jnp.bitwise_and(bits, 0x007FFFFF), 0x3F800000)
    m = lax.bitcast_convert_type(m_bits, jnp.float32)
    s = (m - 1.0) / (m + 1.0)
    s2 = s * s
    p = 2.0 * s * (1.0 + s2 * (1.0 / 3.0 + s2 * (0.2 + s2 * (1.0 / 7.0 + s2 * (1.0 / 9.0)))))
    return e.astype(jnp.float32) * _LN2 + p


def _sigmoid04(x):
    return 0.4 / (1.0 + jnp.exp(-x))


def _popcount(v):
    v = v - jnp.bitwise_and(lax.shift_right_logical(v, 1), 0x55555555)
    v = jnp.bitwise_and(v, 0x33333333) + jnp.bitwise_and(
        lax.shift_right_logical(v, 2), 0x33333333)
    v = jnp.bitwise_and(v + lax.shift_right_logical(v, 4), 0x0F0F0F0F)
    return lax.shift_right_logical(v * 0x01010101, 24)


def _body(uid_hbm, qid_hbm, tbits_hbm, qbits_hbm, slip_hbm, guess_hbm, out_hbm,
          uid_v, qid_v, tb_v, qb_v, slip_v, guess_v, out_v,
          sem_t, sem_q, sem_s, sem_g):
    wid = lax.axis_index("s") * 2 + lax.axis_index("c")
    base = wid * _BPW

    pltpu.sync_copy(uid_hbm.at[pl.ds(base, _BPW)], uid_v)
    pltpu.sync_copy(qid_hbm.at[pl.ds(base, _BPW)], qid_v)

    cp_t = pltpu.async_copy(tbits_hbm.at[uid_v], tb_v, sem_t)
    cp_q = pltpu.async_copy(qbits_hbm.at[qid_v], qb_v, sem_q)
    cp_s = pltpu.async_copy(slip_hbm.at[qid_v], slip_v, sem_s)
    cp_g = pltpu.async_copy(guess_hbm.at[qid_v], guess_v, sem_g)
    cp_t.wait()
    cp_q.wait()
    cp_s.wait()
    cp_g.wait()

    def chunk(i, carry):
        tb = tb_v[pl.ds(i * 16, 16)]
        qb = qb_v[pl.ds(i * 16, 16)]
        bad = jnp.bitwise_and(qb, jnp.bitwise_not(tb))
        k = _popcount(bad)
        n = lax.bitcast_convert_type(jnp.left_shift(127 - k, 23), jnp.float32)
        sraw = slip_v[pl.ds(i * 16, 16)]
        graw = guess_v[pl.ds(i * 16, 16)]
        a = 1.0 - _sigmoid04(sraw)
        g = jnp.maximum(_sigmoid04(graw), 1e-30)
        out_v[pl.ds(i * 16, 16)] = jnp.exp(n * _ln(a) + (1.0 - n) * _ln(g))
        return carry

    lax.fori_loop(0, _BPW // 16, chunk, 0)
    pltpu.sync_copy(out_v, out_hbm.at[pl.ds(base, _BPW)])


@jax.jit
def _dina(uid, qid, theta_t, slip_1d, guess_1d, qtab_t):
    tbits = _pack_tc(theta_t, theta_t.shape[1], 65536)
    qbits = _pack_tc(qtab_t, qtab_t.shape[1], 32768)
    run = pl.kernel(
        _body,
        out_type=jax.ShapeDtypeStruct((_BATCH,), jnp.float32),
        mesh=plsc.VectorSubcoreMesh(core_axis_name="c", subcore_axis_name="s"),
        compiler_params=pltpu.CompilerParams(needs_layout_passes=False),
        scratch_types=[
            pltpu.VMEM((_BPW,), jnp.int32),
            pltpu.VMEM((_BPW,), jnp.int32),
            pltpu.VMEM((_BPW,), jnp.int32),
            pltpu.VMEM((_BPW,), jnp.int32),
            pltpu.VMEM((_BPW,), jnp.float32),
            pltpu.VMEM((_BPW,), jnp.float32),
            pltpu.VMEM((_BPW,), jnp.float32),
            pltpu.SemaphoreType.DMA,
            pltpu.SemaphoreType.DMA,
            pltpu.SemaphoreType.DMA,
            pltpu.SemaphoreType.DMA,
        ],
    )
    return run(uid, qid, tbits, qbits, slip_1d, guess_1d)


def kernel(user_id, question_id, theta_w, slip_w, guess_w, q_table):
    return _dina(
        user_id.astype(jnp.int32),
        question_id.astype(jnp.int32),
        theta_w.T,
        slip_w.reshape(-1),
        guess_w.reshape(-1),
        q_table.T,
    )
